# baseline (speedup 1.0000x reference)
import functools

import jax
import jax.numpy as jnp
from jax import lax
from jax.experimental import pallas as pl
from jax.experimental.pallas import tpu as pltpu
from jax.experimental.pallas import tpu_sc as plsc

NC = 2
NS = 16
NW = NC * NS

CHUNK = 40
DPAD = 1024


def kernel(idx, table):
    B, H = idx.shape
    V, D = table.shape
    n = B * H
    per_w = n // NW
    n_chunks = per_w // CHUNK

    mesh = plsc.VectorSubcoreMesh(
        core_axis_name="c", subcore_axis_name="s",
        num_cores=NC, num_subcores=NS,
    )

    bat_per_w = B // NW
    chunks_per_b = H // CHUNK

    @functools.partial(
        pl.kernel,
        out_type=jax.ShapeDtypeStruct((B, H, D), jnp.float32),
        mesh=mesh,
        scratch_types=[
            pltpu.VMEM((per_w,), jnp.int32),
            pltpu.VMEM((CHUNK, D), jnp.float32),
            pltpu.VMEM((CHUNK, D), jnp.float32),
            pltpu.SemaphoreType.DMA,
            pltpu.SemaphoreType.DMA,
        ],
        compiler_params=pltpu.CompilerParams(use_tc_tiling_on_sc=False),
    )
    def gather_k(idx_hbm, table_hbm, out_hbm, idx_v, buf0, buf1, sem0, sem1):
        wid = lax.axis_index("s") * NC + lax.axis_index("c")
        base = wid * per_w
        bat0 = wid * bat_per_w
        pltpu.sync_copy(idx_hbm.at[pl.ds(base, per_w)], idx_v)

        def gather(i, buf, sem):
            return pltpu.async_copy(
                table_hbm.at[idx_v.at[pl.ds(i * CHUNK, CHUNK)]], buf, sem
            )

        def scatter(i, buf):
            b = bat0 + i // chunks_per_b
            h = (i % chunks_per_b) * CHUNK
            pltpu.sync_copy(buf, out_hbm.at[b].at[pl.ds(h, CHUNK)])

        gather(0, buf0, sem0)

        @pl.loop(0, n_chunks, step=2)
        def _(i):
            gather(i + 1, buf1, sem1)
            pltpu.make_async_copy(table_hbm.at[pl.ds(0, CHUNK)], buf0, sem0).wait()
            scatter(i, buf0)

            @pl.when(i + 2 < n_chunks)
            def _():
                gather(i + 2, buf0, sem0)

            pltpu.make_async_copy(table_hbm.at[pl.ds(0, CHUNK)], buf1, sem1).wait()
            scatter(i + 1, buf1)

    return gather_k(idx.reshape(n).astype(jnp.int32), table)


# speedup vs baseline: 1.5050x; 1.5050x over previous
import functools

import jax
import jax.numpy as jnp
from jax import lax
from jax.experimental import pallas as pl
from jax.experimental.pallas import tpu as pltpu
from jax.experimental.pallas import tpu_sc as plsc

NC = 2
NS = 16
NW = NC * NS
L = 16

CHUNK = 40
NBAND = 8
DPAD = NBAND * 128


def kernel(idx, table):
    B, H = idx.shape
    V, D = table.shape
    n = B * H
    per_w = n // NW
    n_chunks = per_w // CHUNK
    bat_per_w = B // NW
    chunks_per_b = H // CHUNK
    idx_pad = per_w + L

    mesh = plsc.VectorSubcoreMesh(
        core_axis_name="c", subcore_axis_name="s",
        num_cores=NC, num_subcores=NS,
    )

    @functools.partial(
        pl.kernel,
        out_type=jax.ShapeDtypeStruct((B, H, D), jnp.float32),
        mesh=mesh,
        scratch_types=[
            pltpu.VMEM((idx_pad,), jnp.int32),
            pltpu.VMEM((CHUNK, D), jnp.float32),
            pltpu.VMEM((CHUNK, D), jnp.float32),
            pltpu.VMEM((NBAND, 128), jnp.int32),
            pltpu.VMEM((NBAND, 128), jnp.int32),
            pltpu.SemaphoreType.DMA,
            pltpu.SemaphoreType.DMA,
        ],
    )
    def gather_k(idx_hbm, timg_hbm, out_hbm, pidx_v, buf0, buf1, pb0, pb1,
                 sem0, sem1):
        wid = lax.axis_index("s") * NC + lax.axis_index("c")
        base = wid * per_w
        bat0 = wid * bat_per_w
        pltpu.sync_copy(idx_hbm.at[pl.ds(base, per_w)], pidx_v.at[pl.ds(0, per_w)])

        @pl.loop(0, per_w // L)
        def _(k):
            v = pidx_v[pl.ds(k * L, L)]
            pidx_v[pl.ds(k * L, L)] = ((v >> 3) << 6) | (v & 7)

        def fill_bands(i, pb):
            @pl.loop(0, NBAND)
            def _(t):
                @pl.loop(0, (CHUNK + L - 1) // L)
                def _(j):
                    v = pidx_v[pl.ds(i * CHUNK + j * L, L)]
                    pb[t, pl.ds(j * L, L)] = v + 8 * t

        def gather(pb, buf, sem):
            for t in range(NBAND):
                if 128 * (t + 1) <= D:
                    dst = buf.at[:, pl.ds(128 * t, 128)]
                else:
                    start = pl.multiple_of(128 * t + wid * 0, 128)
                    dst = buf.at[:, pl.ds(start, 128)]
                pltpu.async_copy(
                    timg_hbm.at[pb.at[t, pl.ds(0, CHUNK)]],
                    dst,
                    sem,
                )

        def wait_bands(buf, sem):
            for t in range(NBAND):
                pltpu.make_async_copy(
                    timg_hbm.at[pl.ds(0, CHUNK)],
                    buf.at[:, pl.ds(0, 128)],
                    sem,
                ).wait()

        def scatter(i, buf):
            b = bat0 + i // chunks_per_b
            h = (i % chunks_per_b) * CHUNK
            pltpu.sync_copy(buf, out_hbm.at[b].at[pl.ds(h, CHUNK)])

        fill_bands(0, pb0)
        gather(pb0, buf0, sem0)

        @pl.loop(0, n_chunks, step=2)
        def _(i):
            fill_bands(i + 1, pb1)
            gather(pb1, buf1, sem1)
            wait_bands(buf0, sem0)
            scatter(i, buf0)

            @pl.when(i + 2 < n_chunks)
            def _():
                fill_bands(i + 2, pb0)
                gather(pb0, buf0, sem0)

            wait_bands(buf1, sem1)
            scatter(i + 1, buf1)

    tp = jnp.pad(table, ((0, 0), (0, DPAD - D)))
    timg = tp.reshape(V // 8, 8, NBAND, 128).transpose(0, 2, 1, 3).reshape(V * NBAND, 128)
    return gather_k(idx.reshape(n).astype(jnp.int32), timg)


# speedup vs baseline: 1.5071x; 1.0014x over previous
import functools

import jax
import jax.numpy as jnp
from jax import lax
from jax.experimental import pallas as pl
from jax.experimental.pallas import tpu as pltpu
from jax.experimental.pallas import tpu_sc as plsc

NC = 2
NS = 16
NW = NC * NS
L = 16

CHUNK = 40
NBAND = 8
DPAD = NBAND * 128


def kernel(idx, table):
    B, H = idx.shape
    V, D = table.shape
    n = B * H
    per_w = n // NW
    n_chunks = per_w // CHUNK
    bat_per_w = B // NW
    chunks_per_b = H // CHUNK
    idx_pad = per_w + L

    mesh = plsc.VectorSubcoreMesh(
        core_axis_name="c", subcore_axis_name="s",
        num_cores=NC, num_subcores=NS,
    )

    @functools.partial(
        pl.kernel,
        out_type=jax.ShapeDtypeStruct((B, H, D), jnp.float32),
        mesh=mesh,
        scratch_types=[
            pltpu.VMEM((idx_pad,), jnp.int32),
            pltpu.VMEM((CHUNK, D), jnp.float32),
            pltpu.VMEM((CHUNK, D), jnp.float32),
            pltpu.VMEM((NBAND, 128), jnp.int32),
            pltpu.VMEM((NBAND, 128), jnp.int32),
            pltpu.SemaphoreType.DMA,
            pltpu.SemaphoreType.DMA,
        ],
    )
    def gather_k(idx_hbm, timg_hbm, out_hbm, pidx_v, buf0, buf1, pb0, pb1,
                 sem0, sem1):
        wid = lax.axis_index("s") * NC + lax.axis_index("c")
        base = wid * per_w
        bat0 = wid * bat_per_w
        pltpu.sync_copy(idx_hbm.at[pl.ds(base, per_w)], pidx_v.at[pl.ds(0, per_w)])

        @pl.loop(0, per_w // L)
        def _(k):
            v = pidx_v[pl.ds(k * L, L)]
            pidx_v[pl.ds(k * L, L)] = ((v >> 3) << 6) | (v & 7)

        def fill_bands(i, pb):
            for j in range((CHUNK + L - 1) // L):
                v = pidx_v[pl.ds(i * CHUNK + j * L, L)]
                for t in range(NBAND):
                    pb[t, pl.ds(j * L, L)] = v + 8 * t

        def gather(pb, buf, sem):
            for t in range(NBAND):
                if 128 * (t + 1) <= D:
                    dst = buf.at[:, pl.ds(128 * t, 128)]
                else:
                    start = pl.multiple_of(128 * t + wid * 0, 128)
                    dst = buf.at[:, pl.ds(start, 128)]
                pltpu.async_copy(
                    timg_hbm.at[pb.at[t, pl.ds(0, CHUNK)]],
                    dst,
                    sem,
                )

        def wait_bands(buf, sem):
            for t in range(NBAND):
                pltpu.make_async_copy(
                    timg_hbm.at[pl.ds(0, CHUNK)],
                    buf.at[:, pl.ds(0, 128)],
                    sem,
                ).wait()

        def scatter(i, buf):
            b = bat0 + i // chunks_per_b
            h = (i % chunks_per_b) * CHUNK
            pltpu.sync_copy(buf, out_hbm.at[b].at[pl.ds(h, CHUNK)])

        fill_bands(0, pb0)
        gather(pb0, buf0, sem0)

        @pl.loop(0, n_chunks, step=2)
        def _(i):
            fill_bands(i + 1, pb1)
            gather(pb1, buf1, sem1)
            wait_bands(buf0, sem0)
            scatter(i, buf0)

            @pl.when(i + 2 < n_chunks)
            def _():
                fill_bands(i + 2, pb0)
                gather(pb0, buf0, sem0)

            wait_bands(buf1, sem1)
            scatter(i + 1, buf1)

    tp = jnp.pad(table, ((0, 0), (0, DPAD - D)))
    timg = tp.reshape(V // 8, 8, NBAND, 128).transpose(0, 2, 1, 3).reshape(V * NBAND, 128)
    return gather_k(idx.reshape(n).astype(jnp.int32), timg)
